# Initial kernel scaffold; baseline (speedup 1.0000x reference)
import jax
import jax.numpy as jnp
from jax.experimental import pallas as pl


def kernel(x, edge_index, edge_attr, Wq, bq, Wk, bk, Wv, bv, Ws, bs, g1, be1, W1, b1, W2, b2, g2, be2):
    raise NotImplementedError("write your pallas kernel here")


# speedup vs baseline: 28.0219x; 28.0219x over previous
import functools
import math

import jax
import jax.numpy as jnp
from jax import lax
from jax.experimental import pallas as pl
from jax.experimental.pallas import tpu as pltpu, tpu_sc as plsc

N = 10000
E = 320000
D = 128
H = 8
C = 16
FF = 2 * D

NC = 2
NS = 16
NW = NC * NS
EW = E // NW
CH = 80
NCHUNK = EW // CH
EC = E // NC
NP = 10240
RT = NP // NS
ZR = 32

f32 = jnp.float32
i32 = jnp.int32


def _node_tables_body(x_ref, w_ref, b_ref, q_ref, kv_ref, skip_ref):
    y = jnp.dot(x_ref[...], w_ref[...], preferred_element_type=f32) + b_ref[...]
    q_ref[...] = y[:, :D]
    kv_ref[...] = y[:, D:3 * D]
    skip_ref[...] = y[:, 3 * D:]


def _node_tables(x, wall, ball):
    bn = 1000
    return pl.pallas_call(
        _node_tables_body,
        grid=(N // bn,),
        in_specs=[
            pl.BlockSpec((bn, D), lambda i: (i, 0)),
            pl.BlockSpec((D, 4 * D), lambda i: (0, 0)),
            pl.BlockSpec((1, 4 * D), lambda i: (0, 0)),
        ],
        out_specs=[
            pl.BlockSpec((bn, D), lambda i: (i, 0)),
            pl.BlockSpec((bn, 2 * D), lambda i: (i, 0)),
            pl.BlockSpec((bn, D), lambda i: (i, 0)),
        ],
        out_shape=[
            jax.ShapeDtypeStruct((N, D), f32),
            jax.ShapeDtypeStruct((N, 2 * D), f32),
            jax.ShapeDtypeStruct((N, D), f32),
        ],
    )(x, wall, ball)


_sc_mesh = plsc.VectorSubcoreMesh(core_axis_name="c", subcore_axis_name="s")


@functools.partial(
    pl.kernel,
    mesh=_sc_mesh,
    out_type=[
        jax.ShapeDtypeStruct((E, D), f32),
        jax.ShapeDtypeStruct((E, 2 * D), f32),
    ],
    scratch_types=[
        pltpu.VMEM((CH,), i32),
        pltpu.VMEM((CH,), i32),
        pltpu.VMEM((CH, D), f32),
        pltpu.VMEM((CH, 2 * D), f32),
        pltpu.SemaphoreType.DMA,
        pltpu.SemaphoreType.DMA,
    ],
)
def _sc_gather(src_h, dst_h, qtab_h, kvtab_h, qd_out, kv_out,
               sidx, didx, qbuf, kvbuf, sem1, sem2):
    wid = lax.axis_index("s") * NC + lax.axis_index("c")

    def body(i, _):
        base = pl.multiple_of(wid * EW + i * CH, 8)
        pltpu.sync_copy(src_h.at[pl.ds(base, CH)], sidx)
        pltpu.sync_copy(dst_h.at[pl.ds(base, CH)], didx)
        g1 = pltpu.async_copy(qtab_h.at[didx], qbuf, sem1)
        g2 = pltpu.async_copy(kvtab_h.at[sidx], kvbuf, sem2)
        g1.wait()
        g2.wait()
        pltpu.sync_copy(qbuf, qd_out.at[pl.ds(base, CH)])
        pltpu.sync_copy(kvbuf, kv_out.at[pl.ds(base, CH)])
        return 0

    lax.fori_loop(0, NCHUNK, body, 0)


def _edge_body(attr_ref, qd_ref, xkv_ref, wkv_ref, bkv_ref, mh_ref, mht_ref,
               wv_ref, wrep_ref):
    akv = jnp.dot(attr_ref[...], wkv_ref[...], preferred_element_type=f32)
    kv = akv + bkv_ref[...] + xkv_ref[...]
    k = kv[:, :D]
    v = kv[:, D:]
    alpha = jnp.dot(qd_ref[...] * k, mh_ref[...], preferred_element_type=f32) * 0.25
    w = jnp.exp(alpha)
    wrep = jnp.dot(w, mht_ref[...], preferred_element_type=f32)
    wv_ref[...] = wrep * v
    wrep_ref[...] = wrep


def _edge_pass(attr, qd, xkv, wkv, bkv, mh, mht):
    be = 2000
    return pl.pallas_call(
        _edge_body,
        grid=(E // be,),
        in_specs=[
            pl.BlockSpec((be, D), lambda i: (i, 0)),
            pl.BlockSpec((be, D), lambda i: (i, 0)),
            pl.BlockSpec((be, 2 * D), lambda i: (i, 0)),
            pl.BlockSpec((D, 2 * D), lambda i: (0, 0)),
            pl.BlockSpec((1, 2 * D), lambda i: (0, 0)),
            pl.BlockSpec((D, H), lambda i: (0, 0)),
            pl.BlockSpec((H, D), lambda i: (0, 0)),
        ],
        out_specs=[
            pl.BlockSpec((be, D), lambda i: (i, 0)),
            pl.BlockSpec((be, D), lambda i: (i, 0)),
        ],
        out_shape=[
            jax.ShapeDtypeStruct((E, D), f32),
            jax.ShapeDtypeStruct((E, D), f32),
        ],
    )(attr, qd, xkv, wkv, bkv, mh, mht)


def _make_scatter(width):
    @functools.partial(
        pl.kernel,
        mesh=_sc_mesh,
        out_type=jax.ShapeDtypeStruct((NC, NP, width), f32),
        scratch_types=[
            pltpu.VMEM((CH,), i32),
            pltpu.VMEM((CH, width), f32),
            pltpu.VMEM((ZR, width), f32),
            pltpu.VMEM_SHARED((NP, width), f32),
        ],
    )
    def _sc_scatter(dst_h, val_h, acc_out, didx, vbuf, zv, acc_sh):
        cid = lax.axis_index("c")
        sid = lax.axis_index("s")
        zero = jnp.zeros((C,), f32)

        def zbody(i, _):
            for j in range(width // C):
                zv[i, pl.ds(j * C, C)] = zero
            return 0

        lax.fori_loop(0, ZR, zbody, 0)

        def zcopy(i, _):
            r = pl.multiple_of(sid * RT + i * ZR, 8)
            pltpu.sync_copy(zv, acc_sh.at[pl.ds(r, ZR)])
            return 0

        lax.fori_loop(0, RT // ZR, zcopy, 0)
        plsc.subcore_barrier()

        def body(i, _):
            base = pl.multiple_of(cid * EC + sid * EW + i * CH, 8)
            pltpu.sync_copy(dst_h.at[pl.ds(base, CH)], didx)
            pltpu.sync_copy(val_h.at[pl.ds(base, CH)], vbuf)
            pltpu.sync_copy(vbuf, acc_sh.at[didx], add=True)
            return 0

        lax.fori_loop(0, NCHUNK, body, 0)
        plsc.subcore_barrier()

        r = pl.multiple_of(sid * RT, 8)
        pltpu.sync_copy(acc_sh.at[pl.ds(r, RT)], acc_out.at[cid, pl.ds(r, RT)])

    return _sc_scatter


_sc_scatter_v = _make_scatter(D)


def _node_out_body(accv_ref, accw_ref, skip_ref, x_ref,
                   w1_ref, b1_ref, w2_ref, b2_ref,
                   g1_ref, be1_ref, g2_ref, be2_ref, out_ref):
    av = accv_ref[0] + accv_ref[1]
    srep = accw_ref[0] + accw_ref[1]
    out = av / (srep + 1e-16) + skip_ref[...]
    h = x_ref[...] + out
    mu = jnp.mean(h, axis=1, keepdims=True)
    var = jnp.mean((h - mu) ** 2, axis=1, keepdims=True)
    h = (h - mu) / jnp.sqrt(var + 1e-5) * g1_ref[...] + be1_ref[...]
    ff = jnp.dot(
        jnp.maximum(jnp.dot(h, w1_ref[...], preferred_element_type=f32)
                    + b1_ref[...], 0.0),
        w2_ref[...], preferred_element_type=f32) + b2_ref[...]
    h = h + ff
    mu = jnp.mean(h, axis=1, keepdims=True)
    var = jnp.mean((h - mu) ** 2, axis=1, keepdims=True)
    out_ref[...] = (h - mu) / jnp.sqrt(var + 1e-5) * g2_ref[...] + be2_ref[...]


def _node_out(accv, accw, skip, x, w1, b1, w2, b2, g1, be1, g2, be2):
    bn = 1000
    return pl.pallas_call(
        _node_out_body,
        grid=(N // bn,),
        in_specs=[
            pl.BlockSpec((NC, bn, D), lambda i: (0, i, 0)),
            pl.BlockSpec((NC, bn, D), lambda i: (0, i, 0)),
            pl.BlockSpec((bn, D), lambda i: (i, 0)),
            pl.BlockSpec((bn, D), lambda i: (i, 0)),
            pl.BlockSpec((D, FF), lambda i: (0, 0)),
            pl.BlockSpec((1, FF), lambda i: (0, 0)),
            pl.BlockSpec((FF, D), lambda i: (0, 0)),
            pl.BlockSpec((1, D), lambda i: (0, 0)),
            pl.BlockSpec((1, D), lambda i: (0, 0)),
            pl.BlockSpec((1, D), lambda i: (0, 0)),
            pl.BlockSpec((1, D), lambda i: (0, 0)),
            pl.BlockSpec((1, D), lambda i: (0, 0)),
        ],
        out_specs=pl.BlockSpec((bn, D), lambda i: (i, 0)),
        out_shape=jax.ShapeDtypeStruct((N, D), f32),
    )(accv, accw, skip, x, w1, b1, w2, b2, g1, be1, g2, be2)


def kernel(x, edge_index, edge_attr, Wq, bq, Wk, bk, Wv, bv, Ws, bs,
           g1, be1, W1, b1, W2, b2, g2, be2):
    zb = jnp.zeros_like(bk)
    wall = jnp.concatenate([Wq, Wk, Wv, Ws], axis=1)
    ball = jnp.concatenate([bq, zb, zb, bs])[None, :]
    wkv = jnp.concatenate([Wk, Wv], axis=1)
    bkv = jnp.concatenate([bk, bv])[None, :]

    head = jnp.arange(D, dtype=i32) // C
    mh = (head[:, None] == jnp.arange(H, dtype=i32)[None, :]).astype(f32)
    mht = mh.T

    src = edge_index[0]
    dst = edge_index[1]

    qtab, kvtab, skip = _node_tables(x, wall, ball)
    qd, xkv = _sc_gather(src, dst, qtab, kvtab)
    wv, wrep = _edge_pass(edge_attr, qd, xkv, wkv, bkv, mh, mht)
    accv = _sc_scatter_v(dst, wv)
    accw = _sc_scatter_v(dst, wrep)
    return _node_out(accv, accw, skip, x,
                     W1, b1[None, :], W2, b2[None, :],
                     g1[None, :], be1[None, :], g2[None, :], be2[None, :])
